# baseline (speedup 1.0000x reference)
import functools

import jax
import jax.numpy as jnp
from jax import lax
from jax.experimental import pallas as pl
from jax.experimental.pallas import tpu as pltpu
from jax.experimental.pallas import tpu_sc as plsc

_B = 2
_N = 2048
_K = 6
_D = 128
_R = _B * _N
_E = _R * _K
_H1 = 514
_M = 256
_BLK = 1024
_NW = 32
_EPW = _E // _NW


def _sigmoid(x):
    return 0.5 * jnp.tanh(x * 0.5) + 0.5


def _silu(x):
    return x * _sigmoid(x)


def _knn_body(ctb_ref, ct_ref, idx_ref, dk_ref):
    b = pl.program_id(0)
    ctb = ctb_ref[0]
    ct = ct_ref[0]
    cb = jnp.transpose(ctb)
    cross = jnp.dot(cb, ct, preferred_element_type=jnp.float32)
    ni = jnp.sum(cb * cb, axis=1, keepdims=True)
    nj = jnp.sum(ct * ct, axis=0, keepdims=True)
    dist = jnp.maximum(ni + nj - 2.0 * cross, 0.0)
    col = lax.broadcasted_iota(jnp.int32, (_BLK, _N), 1)
    q = ((lax.bitcast_convert_type(dist, jnp.int32)
          & jnp.int32(-2048)) | col)
    ds, ids = [], []
    for k in range(_K):
        m = jnp.min(q, axis=1, keepdims=True)
        ids.append(m & jnp.int32(2047))
        ds.append(lax.bitcast_convert_type(m & jnp.int32(-2048), jnp.float32))
        if k + 1 < _K:
            q = jnp.where(q == m, jnp.int32(2147483647), q)
    dk_ref[...] = jnp.transpose(jnp.concatenate(ds, axis=1))
    idx_ref[...] = jnp.transpose(jnp.concatenate(ids, axis=1)) + b * _N


def _knn(coors):
    coors_t = jnp.transpose(coors, (0, 2, 1))
    idxf, dk = pl.pallas_call(
        _knn_body,
        grid=(_B, _N // _BLK),
        in_specs=[
            pl.BlockSpec((1, 3, _BLK), lambda b, r: (b, 0, r)),
            pl.BlockSpec((1, 3, _N), lambda b, r: (b, 0, 0)),
        ],
        out_specs=[
            pl.BlockSpec((_K, _BLK),
                         lambda b, r: (0, b * (_N // _BLK) + r)),
            pl.BlockSpec((_K, _BLK),
                         lambda b, r: (0, b * (_N // _BLK) + r)),
        ],
        out_shape=[
            jax.ShapeDtypeStruct((_K, _R), jnp.int32),
            jax.ShapeDtypeStruct((_K, _R), jnp.float32),
        ],
    )(coors_t, coors_t)
    return idxf, dk


@functools.lru_cache(maxsize=2)
def _make_sc_gather(d):
    @functools.partial(
        pl.kernel,
        out_type=jax.ShapeDtypeStruct((_E, d), jnp.float32),
        mesh=plsc.VectorSubcoreMesh(core_axis_name="c", subcore_axis_name="s"),
        scratch_types=[
            pltpu.VMEM((_EPW,), jnp.int32),
            pltpu.VMEM((_EPW, d), jnp.float32),
            pltpu.SemaphoreType.DMA,
        ],
    )
    def _sc_gather_kernel(table_hbm, idx_hbm, out_hbm, idx_v, rows_v, sem):
        wid = lax.axis_index("s") * 2 + lax.axis_index("c")
        base = wid * _EPW
        pltpu.sync_copy(idx_hbm.at[pl.ds(base, _EPW)], idx_v)
        pltpu.async_copy(table_hbm.at[idx_v], rows_v, sem).wait()
        pltpu.sync_copy(rows_v, out_hbm.at[pl.ds(base, _EPW)])

    return _sc_gather_kernel


def _sc_gather(table, idx):
    return _make_sc_gather(table.shape[1])(table, idx)


def _layer_body(f_ref, g_ref, dk_ref, w1i_ref, w1j_ref, w1d_ref, be1_ref,
                w2_ref, be2_ref, wg_ref, bg_ref, wn1a_ref, wn1b_ref,
                bn1_ref, wn2_ref, bn2_ref, out_ref):
    bf = jnp.bfloat16
    fi = f_ref[...]
    f = fi if fi.shape[1] == _D else jnp.concatenate([fi, fi], axis=1)
    fb = fi.astype(bf)
    w1i = w1i_ref[...]
    w1j = w1j_ref[...]
    w1d = w1d_ref[...]
    be1 = be1_ref[...]
    w2 = w2_ref[...]
    be2 = be2_ref[...]
    wg = wg_ref[...]
    bg = bg_ref[0, 0]
    pre_i = jnp.dot(fb, w1i, preferred_element_type=jnp.float32) + be1
    acc = jnp.zeros((_BLK, _M), jnp.float32)
    dkt = jnp.transpose(dk_ref[...])
    for k in range(_K):
        g = g_ref[k].astype(bf)
        dk = dkt[:, k:k + 1]
        pre = (jnp.dot(g, w1j, preferred_element_type=jnp.float32)
               + pre_i + dk * w1d)
        h = _silu(pre.astype(bf))
        m = _silu((jnp.dot(h, w2, preferred_element_type=jnp.float32)
                   + be2).astype(bf))
        gate_pre = jnp.sum(m * wg, axis=1, keepdims=True) + bg
        acc = acc + (m * _sigmoid(gate_pre)).astype(jnp.float32)
    n_pre = (jnp.dot(fi, wn1a_ref[...], preferred_element_type=jnp.float32)
             + jnp.dot(acc, wn1b_ref[...],
                       preferred_element_type=jnp.float32)
             + bn1_ref[...])
    n1 = _silu(n_pre.astype(bf))
    out_ref[...] = (jnp.dot(n1, wn2_ref[...], preferred_element_type=jnp.float32)
                    + bn2_ref[...] + f)


def _layer(feats, g3, dk3, w1i, w1j, w1d, be1, w2, be2, wgt, bg,
           wn1a, wn1b, bn1, wn2, bn2):
    din = feats.shape[1]
    wspec = lambda shp: pl.BlockSpec(shp, lambda r: tuple(0 for _ in shp))
    return pl.pallas_call(
        _layer_body,
        grid=(_R // _BLK,),
        in_specs=[
            pl.BlockSpec((_BLK, din), lambda r: (r, 0)),
            pl.BlockSpec((_K, _BLK, din), lambda r: (0, r, 0)),
            pl.BlockSpec((_K, _BLK), lambda r: (0, r)),
            wspec((din, _H1)),
            wspec((din, _H1)),
            wspec((1, _H1)),
            wspec((1, _H1)),
            wspec((_H1, _M)),
            wspec((1, _M)),
            wspec((1, _M)),
            wspec((1, 1)),
            wspec((din, _M)),
            wspec((_M, _M)),
            wspec((1, _M)),
            wspec((_M, _D)),
            wspec((1, _D)),
        ],
        out_specs=pl.BlockSpec((_BLK, _D), lambda r: (r, 0)),
        out_shape=jax.ShapeDtypeStruct((_R, _D), jnp.float32),
    )(feats, g3, dk3, w1i, w1j, w1d, be1, w2, be2, wgt, bg,
      wn1a, wn1b, bn1, wn2, bn2)


def _pool_body(f_ref, wm1_ref, bm1_ref, wm2_ref, bm2_ref, out_ref):
    f = f_ref[...]
    ps = []
    for b in range(_B):
        ps.append(jnp.sum(f[b * _N:(b + 1) * _N], axis=0, keepdims=True)
                  * (1.0 / _N))
    pooled = jnp.concatenate(ps, axis=0)
    h = jnp.maximum(
        jnp.dot(pooled, wm1_ref[...], preferred_element_type=jnp.float32)
        + bm1_ref[...], 0.0)
    out_ref[...] = (jnp.dot(h, wm2_ref[...], preferred_element_type=jnp.float32)
                    + bm2_ref[...])


def _pool(feats, wm1, bm1, wm2, bm2):
    wspec = lambda shp: pl.BlockSpec(shp, lambda: tuple(0 for _ in shp))
    return pl.pallas_call(
        _pool_body,
        in_specs=[
            wspec((_R, _D)),
            wspec((_D, _M)),
            wspec((1, _M)),
            wspec((_M, _D)),
            wspec((1, _D)),
        ],
        out_specs=wspec((_B, _D)),
        out_shape=jax.ShapeDtypeStruct((_B, _D), jnp.float32),
    )(feats, wm1, bm1, wm2, bm2)


def kernel(x, context, mask, We1, be1, We2, be2, Wg, bg, Wn1, bn1, Wn2, bn2,
           Wm1, bm1, Wm2, bm2):
    feats = jnp.concatenate([x, x], axis=-1).reshape(_R, _D)

    idxf, dkf = _knn(context)
    idx_kr = idxf.reshape(_E)

    for l in range(We1.shape[0]):
        w1i = We1[l, :_D]
        w1j = We1[l, _D:2 * _D]
        wn1a = Wn1[l, :_D]
        din = feats.shape[1]
        g = _sc_gather(feats, idx_kr)
        g3 = g.reshape(_K, _R, din)
        feats = _layer(
            feats, g3, dkf,
            w1i.astype(jnp.bfloat16),
            w1j.astype(jnp.bfloat16),
            We1[l, 2 * _D:2 * _D + 1],
            be1[l][None, :],
            We2[l].astype(jnp.bfloat16),
            be2[l][None, :],
            Wg[l].reshape(1, _M), bg[l].reshape(1, 1),
            wn1a, Wn1[l, _D:], bn1[l][None, :],
            Wn2[l], bn2[l][None, :],
        )

    out = _pool(feats, Wm1, bm1[None, :], Wm2, bm2[None, :])
    return jnp.tile(out[:, None, :], (1, _N, 1))
